# baseline (speedup 1.0000x reference)
import jax
import jax.numpy as jnp
from jax import lax
from jax.experimental import pallas as pl
from jax.experimental.pallas import tpu as pltpu
from jax.experimental.pallas import tpu_sc as plsc

_N = 100000
_D = 128
_B = 64

_L = 16
_NW = 32
_RPW = 3200
_NPAD = _NW * _RPW
_CHUNK = 128


def _sc_body(flat_hbm, batch_hbm, state_hbm, pae_hbm, nac_hbm,
             batch_v, state_v, eidx_v, nidx_v, pae_v, nac_v, esem, nsem):
    cid = lax.axis_index("c")
    sid = lax.axis_index("s")
    wid = sid * 2 + cid
    base = wid * _RPW

    pltpu.sync_copy(batch_hbm.at[pl.ds(base, _RPW)], batch_v)
    pltpu.sync_copy(state_hbm, state_v)

    def build_indices(k, carry):
        lanes = lax.broadcasted_iota(jnp.int32, (_L,), 0)
        r_loc = k * _L + lanes
        r_cl = jnp.minimum(base + r_loc, _N - 1)
        b = batch_v[pl.ds(k * _L, _L)]
        c = plsc.load_gather(state_v, [b])
        row0 = r_cl * _D
        eidx_v[pl.ds(k * _L, _L)] = row0 + c
        for j in range(3):
            plsc.store_scatter(nidx_v, [r_loc * 3 + j], row0 + (2 + j))
        return carry

    lax.fori_loop(0, _RPW // _L, build_indices, 0)

    handles = []
    for g in range(_RPW // _CHUNK):
        handles.append(pltpu.async_copy(
            flat_hbm.at[eidx_v.at[pl.ds(g * _CHUNK, _CHUNK)]],
            pae_v.at[pl.ds(g * _CHUNK, _CHUNK)], esem))
    for g in range(3 * _RPW // _CHUNK):
        handles.append(pltpu.async_copy(
            flat_hbm.at[nidx_v.at[pl.ds(g * _CHUNK, _CHUNK)]],
            nac_v.at[pl.ds(g * _CHUNK, _CHUNK)], nsem))
    for h in handles:
        h.wait()

    pltpu.sync_copy(pae_v, pae_hbm.at[pl.ds(base, _RPW)])
    pltpu.sync_copy(nac_v, nac_hbm.at[pl.ds(base * 3, 3 * _RPW)])


def _make_sc_call():
    mesh = plsc.VectorSubcoreMesh(core_axis_name="c", subcore_axis_name="s")
    return pl.kernel(
        _sc_body,
        mesh=mesh,
        compiler_params=pltpu.CompilerParams(needs_layout_passes=False),
        out_type=(
            jax.ShapeDtypeStruct((_NPAD,), jnp.float32),
            jax.ShapeDtypeStruct((3 * _NPAD,), jnp.float32),
        ),
        scratch_types=[
            pltpu.VMEM((_RPW,), jnp.int32),
            pltpu.VMEM((128,), jnp.int32),
            pltpu.VMEM((_RPW,), jnp.int32),
            pltpu.VMEM((3 * _RPW,), jnp.int32),
            pltpu.VMEM((_RPW,), jnp.float32),
            pltpu.VMEM((3 * _RPW,), jnp.float32),
            pltpu.SemaphoreType.DMA,
            pltpu.SemaphoreType.DMA,
        ],
    )


def kernel(node_features, batch, state):
    flat = node_features.reshape(_N * _D)
    batch_i = batch.astype(jnp.int32)
    state_i = jnp.concatenate(
        [state.astype(jnp.int32), jnp.zeros((128 - _B,), jnp.int32)])
    batch_pad = jnp.concatenate(
        [batch_i, jnp.zeros((_NPAD - _N,), jnp.int32)])
    pae_flat, nac_flat = _make_sc_call()(flat, batch_pad, state_i)
    per_atom_energy = pae_flat[:_N].reshape(_N, 1)
    nac = nac_flat.reshape(_NPAD, 3)[:_N]
    return (per_atom_energy, nac)


# speedup vs baseline: 1.3273x; 1.3273x over previous
import jax
import jax.numpy as jnp
from jax import lax
from jax.experimental import pallas as pl
from jax.experimental.pallas import tpu as pltpu
from jax.experimental.pallas import tpu_sc as plsc

_N = 100000
_D = 128
_B = 64
_G = 8

_L = 16
_NW = 32
_RPW = 3200
_TAIL = _N - (_NW - 1) * _RPW
_CHUNK = 128


def _sc_body(gran_hbm, batch_hbm, state_hbm, pae_hbm, nac_hbm,
             batch_v, state_v, eidx_v, nidx_v, c15_v, rows_v,
             pae_v, nac_v, sem):
    cid = lax.axis_index("c")
    sid = lax.axis_index("s")
    wid = sid * 2 + cid
    base = wid * _RPW
    is_tail = wid == _NW - 1

    pltpu.sync_copy(state_hbm, state_v)

    @pl.when(jnp.logical_not(is_tail))
    def _():
        pltpu.sync_copy(batch_hbm.at[pl.ds(base, _RPW)], batch_v)

    @pl.when(is_tail)
    def _():
        pltpu.sync_copy(batch_hbm.at[pl.ds(base, _TAIL)],
                        batch_v.at[pl.ds(0, _TAIL)])

    def build_indices(k, carry):
        lanes = lax.broadcasted_iota(jnp.int32, (_L,), 0)
        r_cl = jnp.minimum(base + k * _L + lanes, _N - 1)
        b = jnp.clip(batch_v[pl.ds(k * _L, _L)], 0, _B - 1)
        c = plsc.load_gather(state_v, [b])
        gr = r_cl * _G
        eidx_v[pl.ds(k * _L, _L)] = gr + (c >> 4)
        c15_v[pl.ds(k * _L, _L)] = c & (_L - 1)
        nidx_v[pl.ds(k * _L, _L)] = gr
        return carry

    lax.fori_loop(0, _RPW // _L, build_indices, 0)

    handles = []
    for g in range(_RPW // _CHUNK):
        handles.append(pltpu.async_copy(
            gran_hbm.at[eidx_v.at[pl.ds(g * _CHUNK, _CHUNK)]],
            rows_v.at[pl.ds(g * _CHUNK, _CHUNK), :], sem))
    for h in handles:
        h.wait()

    def extract_energy(k, carry):
        lanes = lax.broadcasted_iota(jnp.int32, (_L,), 0)
        r_loc = k * _L + lanes
        pae_v[pl.ds(k * _L, _L)] = plsc.load_gather(
            rows_v, [r_loc, c15_v[pl.ds(k * _L, _L)]])
        return carry

    lax.fori_loop(0, _RPW // _L, extract_energy, 0)

    handles = []
    for g in range(_RPW // _CHUNK):
        handles.append(pltpu.async_copy(
            gran_hbm.at[nidx_v.at[pl.ds(g * _CHUNK, _CHUNK)]],
            rows_v.at[pl.ds(g * _CHUNK, _CHUNK), :], sem))
    for h in handles:
        h.wait()

    def extract_nac(k, carry):
        lanes = lax.broadcasted_iota(jnp.int32, (_L,), 0)
        r_loc = k * _L + lanes
        for j in range(3):
            vj = plsc.load_gather(rows_v, [r_loc, lanes * 0 + (2 + j)])
            plsc.store_scatter(nac_v, [r_loc, lanes * 0 + j], vj)
        return carry

    lax.fori_loop(0, _RPW // _L, extract_nac, 0)

    @pl.when(jnp.logical_not(is_tail))
    def _():
        pltpu.sync_copy(pae_v, pae_hbm.at[pl.ds(base, _RPW)])
        pltpu.sync_copy(nac_v, nac_hbm.at[pl.ds(base, _RPW), :])

    @pl.when(is_tail)
    def _():
        pltpu.sync_copy(pae_v.at[pl.ds(0, _TAIL)],
                        pae_hbm.at[pl.ds(base, _TAIL)])
        pltpu.sync_copy(nac_v.at[pl.ds(0, _TAIL), :],
                        nac_hbm.at[pl.ds(base, _TAIL), :])


def _make_sc_call():
    mesh = plsc.VectorSubcoreMesh(core_axis_name="c", subcore_axis_name="s")
    return pl.kernel(
        _sc_body,
        mesh=mesh,
        compiler_params=pltpu.CompilerParams(
            needs_layout_passes=False, use_tc_tiling_on_sc=False),
        out_type=(
            jax.ShapeDtypeStruct((_N,), jnp.float32),
            jax.ShapeDtypeStruct((_N, 3), jnp.float32),
        ),
        scratch_types=[
            pltpu.VMEM((_RPW,), jnp.int32),
            pltpu.VMEM((128,), jnp.int32),
            pltpu.VMEM((_RPW,), jnp.int32),
            pltpu.VMEM((_RPW,), jnp.int32),
            pltpu.VMEM((_RPW,), jnp.int32),
            pltpu.VMEM((_RPW, _L), jnp.float32),
            pltpu.VMEM((_RPW,), jnp.float32),
            pltpu.VMEM((_RPW, 3), jnp.float32),
            pltpu.SemaphoreType.DMA,
        ],
    )


def kernel(node_features, batch, state):
    gran = node_features.reshape(_N * _G, _L)
    batch_i = batch.astype(jnp.int32)
    state_pad = jnp.concatenate(
        [state.astype(jnp.int32), jnp.zeros((128 - _B,), jnp.int32)])
    pae, nac = _make_sc_call()(gran, batch_i, state_pad)
    return (pae.reshape(_N, 1), nac)
